# baseline (speedup 1.0000x reference)
import dataclasses
import functools

import jax
import jax.numpy as jnp
from jax import lax
from jax.experimental import pallas as pl
from jax.experimental.pallas import tpu as pltpu
from jax.experimental.pallas import tpu_sc as plsc

N = 10000
E = 320000
D = 128
H = 128
GATE_IN = D + 2 * H

NC = 2
NS = 16
NW = NC * NS
L = 16

B = 128
NCH = 80
NP = 2
NCH2 = NCH // NP
EPT = NCH * B
EPT2 = EPT // NP
E_PAD = NW * EPT
CPR = 624
TAIL = N - NS * CPR

@functools.cache
def _sc_kernels():
    mesh = plsc.VectorSubcoreMesh(core_axis_name="c", subcore_axis_name="s",
                                  num_cores=NC, num_subcores=NS)
    cp = pltpu.CompilerParams()
    if "needs_layout_passes" in pltpu.CompilerParams.__dataclass_fields__:
        cp = dataclasses.replace(cp, needs_layout_passes=False)
    deg = functools.partial(
        pl.kernel,
        compiler_params=cp,
        out_type=jax.ShapeDtypeStruct((NW, N), jnp.float32),
        mesh=mesh,
        scratch_types=[
            pltpu.VMEM((EPT,), jnp.int32),
            pltpu.VMEM((EPT,), jnp.float32),
            pltpu.VMEM((N,), jnp.float32),
        ],
    )(_deg_body)
    msg = functools.partial(
        pl.kernel,
        compiler_params=cp,
        out_type=jax.ShapeDtypeStruct((NC, N, H), jnp.float32),
        mesh=mesh,
        scratch_types=[
            pltpu.VMEM((NCH2, B), jnp.int32),
            pltpu.VMEM((NCH2, B), jnp.int32),
            pltpu.VMEM((EPT2,), jnp.float32),
            pltpu.VMEM((B, H), jnp.float32),
            pltpu.VMEM((B, H), jnp.float32),
            pltpu.VMEM_SHARED((N, H), jnp.float32),
            pltpu.SemaphoreType.DMA,
            pltpu.SemaphoreType.DMA,
            pltpu.SemaphoreType.DMA,
            pltpu.SemaphoreType.DMA,
        ],
    )(_msg_body)
    return deg, msg


def _deg_body(dst_hbm, ew_hbm, out_hbm, dst_v, ew_v, deg_v):
    cid = lax.axis_index("c")
    sid = lax.axis_index("s")
    wid = sid * NC + cid

    @pl.loop(0, N, step=L)
    def _(i):
        deg_v.at[pl.ds(i, L)][...] = jnp.zeros((L,), jnp.float32)

    pltpu.sync_copy(dst_hbm.at[wid], dst_v)
    pltpu.sync_copy(ew_hbm.at[wid], ew_v)

    @pl.loop(0, EPT, step=L)
    def _(i):
        idx = dst_v.at[pl.ds(i, L)][...]
        val = ew_v.at[pl.ds(i, L)][...]
        plsc.addupdate_scatter(deg_v, [idx], val)

    pltpu.sync_copy(deg_v, out_hbm.at[wid])


def _msg_body(xws_hbm, src_hbm, dst_hbm, ew_hbm, out_hbm,
              srcv, dstv, ewv, gbuf0, gbuf1, acc, gs0, gs1, ss0, ss1):
    cid = lax.axis_index("c")
    sid = lax.axis_index("s")
    wid = sid * NC + cid

    @pl.loop(0, B)
    def _(r):
        for j in range(H // L):
            gbuf0.at[r, pl.ds(j * L, L)][...] = jnp.zeros((L,), jnp.float32)

    r0 = sid * CPR
    for k in range(CPR // B):
        pltpu.sync_copy(gbuf0, acc.at[pl.ds(r0 + k * B, B)])
    rem = CPR - (CPR // B) * B
    pltpu.sync_copy(gbuf0.at[pl.ds(0, rem)],
                    acc.at[pl.ds(r0 + (CPR // B) * B, rem)])

    @pl.when(sid == 0)
    def _():
        pltpu.sync_copy(gbuf0.at[pl.ds(0, TAIL)],
                        acc.at[pl.ds(NS * CPR, TAIL)])

    def scale(buf, ch):
        @pl.loop(0, B, step=L)
        def _(rb):
            ews = ewv.at[pl.ds(ch * B + rb, L)][...]
            for k in range(L):
                s = ews[k]
                for j in range(H // L):
                    r = rb + k
                    buf.at[r, pl.ds(j * L, L)][...] = (
                        buf.at[r, pl.ds(j * L, L)][...] * s)

    def wait_gather(buf, sem):
        pltpu.make_async_copy(xws_hbm.at[srcv.at[0]], buf, sem).wait()

    def wait_scatter(sem):
        pltpu.make_async_copy(gbuf0, acc.at[dstv.at[0]], sem).wait()

    _PROBE_ONLY_CORE = 1
    plsc.subcore_barrier()

    @pl.when(cid == _PROBE_ONLY_CORE)
    def _probe():
      for p in range(NP):
        pltpu.sync_copy(src_hbm.at[wid * NP + p], srcv)
        pltpu.sync_copy(dst_hbm.at[wid * NP + p], dstv)
        pltpu.sync_copy(ew_hbm.at[wid * NP + p], ewv)

        pltpu.async_copy(xws_hbm.at[srcv.at[0]], gbuf0, gs0)

        @pl.loop(0, NCH2, step=2)
        def _(i):
            @pl.when(i > 0)
            def _():
                wait_scatter(ss1)
            pltpu.async_copy(xws_hbm.at[srcv.at[i + 1]], gbuf1, gs1)
            wait_gather(gbuf0, gs0)
            scale(gbuf0, i)
            pltpu.async_copy(gbuf0, acc.at[dstv.at[i]], ss0, add=True)
            wait_gather(gbuf1, gs1)
            scale(gbuf1, i + 1)
            pltpu.async_copy(gbuf1, acc.at[dstv.at[i + 1]], ss1, add=True)

            @pl.when(i + 2 < NCH2)
            def _():
                wait_scatter(ss0)
                pltpu.async_copy(xws_hbm.at[srcv.at[i + 2]], gbuf0, gs0)

        wait_scatter(ss0)
        wait_scatter(ss1)

    plsc.subcore_barrier()
    pltpu.sync_copy(acc.at[pl.ds(r0, CPR)], out_hbm.at[cid, pl.ds(r0, CPR)])

    @pl.when(sid == 0)
    def _():
        pltpu.sync_copy(acc.at[pl.ds(NS * CPR, TAIL)],
                        out_hbm.at[cid, pl.ds(NS * CPR, TAIL)])


_BN = 1000


def _pre_body(x_ref, wt_ref, degp_ref, xws_ref, st_ref):
    xw = lax.dot_general(x_ref[...], wt_ref[...], (((1,), (0,)), ((), ())),
                         preferred_element_type=jnp.float32)
    deg = jnp.sum(degp_ref[...], axis=1) + 1.0
    dinv = lax.rsqrt(deg)
    xws_ref[...] = dinv[:, None] * xw
    st_ref[...] = (dinv * dinv)[:, None] * xw


def _pre_tc(x, w_t, degp):
    return pl.pallas_call(
        _pre_body,
        grid=(N // _BN,),
        in_specs=[
            pl.BlockSpec((_BN, D), lambda i: (i, 0)),
            pl.BlockSpec((D, H), lambda i: (0, 0)),
            pl.BlockSpec((_BN, NW), lambda i: (i, 0)),
        ],
        out_specs=[
            pl.BlockSpec((_BN, H), lambda i: (i, 0)),
            pl.BlockSpec((_BN, H), lambda i: (i, 0)),
        ],
        out_shape=[
            jax.ShapeDtypeStruct((N, H), jnp.float32),
            jax.ShapeDtypeStruct((N, H), jnp.float32),
        ],
    )(x, w_t, degp)


def _fin_body(x_ref, h_ref, c_ref, acc_ref, degp_ref, st_ref, bg_ref,
              wall_ref, ball_ref, hn_ref, cn_ref):
    accsum = acc_ref[0] + acc_ref[1]
    deg = jnp.sum(degp_ref[...], axis=1) + 1.0
    dinv = lax.rsqrt(deg)
    g = jax.nn.sigmoid(dinv[:, None] * accsum + st_ref[...] + bg_ref[...])
    zu = jnp.concatenate([x_ref[...], g, h_ref[...]], axis=1)
    pre = lax.dot_general(zu, wall_ref[...], (((1,), (0,)), ((), ())),
                          preferred_element_type=jnp.float32)
    pre = pre + ball_ref[...]
    f_t = jax.nn.sigmoid(pre[:, 0:H])
    i_t = jax.nn.sigmoid(pre[:, H:2 * H])
    o_t = jax.nn.sigmoid(pre[:, 2 * H:3 * H])
    c_t = jnp.tanh(pre[:, 3 * H:4 * H])
    c_next = f_t * c_ref[...] + i_t * c_t
    hn_ref[...] = o_t * jnp.tanh(c_next)
    cn_ref[...] = c_next


def _fin_tc(x, h, c, acc2, degp, st, bg, wall, ball):
    return pl.pallas_call(
        _fin_body,
        grid=(N // _BN,),
        in_specs=[
            pl.BlockSpec((_BN, D), lambda i: (i, 0)),
            pl.BlockSpec((_BN, H), lambda i: (i, 0)),
            pl.BlockSpec((_BN, H), lambda i: (i, 0)),
            pl.BlockSpec((NC, _BN, H), lambda i: (0, i, 0)),
            pl.BlockSpec((_BN, NW), lambda i: (i, 0)),
            pl.BlockSpec((_BN, H), lambda i: (i, 0)),
            pl.BlockSpec((1, H), lambda i: (0, 0)),
            pl.BlockSpec((GATE_IN, 4 * H), lambda i: (0, 0)),
            pl.BlockSpec((1, 4 * H), lambda i: (0, 0)),
        ],
        out_specs=[
            pl.BlockSpec((_BN, H), lambda i: (i, 0)),
            pl.BlockSpec((_BN, H), lambda i: (i, 0)),
        ],
        out_shape=[
            jax.ShapeDtypeStruct((N, H), jnp.float32),
            jax.ShapeDtypeStruct((N, H), jnp.float32),
        ],
    )(x, h, c, acc2, degp, st, bg, wall, ball)


def kernel(x, edge_index, edge_weight, h, c,
           W_gcn, b_gcn, W_f, b_f, W_i, b_i, W_o, b_o, W_c, b_c):
    src = edge_index[0]
    dst = edge_index[1]
    pad = E_PAD - E
    zpad_i = jnp.zeros((pad,), src.dtype)
    zpad_f = jnp.zeros((pad,), edge_weight.dtype)
    src_p = jnp.concatenate([src, zpad_i])
    dst_p = jnp.concatenate([dst, zpad_i])
    ew_p = jnp.concatenate([edge_weight, zpad_f])

    deg_sc, msg_sc = _sc_kernels()
    degp = deg_sc(dst_p.reshape(NW, EPT), ew_p.reshape(NW, EPT))
    degp = jnp.swapaxes(degp, 0, 1)
    xws, st = _pre_tc(x, W_gcn.T, degp)
    acc2 = msg_sc(xws, src_p.reshape(NW * NP, NCH2, B),
                  dst_p.reshape(NW * NP, NCH2, B), ew_p.reshape(NW * NP, EPT2))

    wall = jnp.concatenate([W_f.T, W_i.T, W_o.T, W_c.T], axis=1)
    ball = jnp.concatenate([b_f, b_i, b_o, b_c]).reshape(1, 4 * H)
    return _fin_tc(x, h, c, acc2, degp, st, b_gcn.reshape(1, H), wall, ball)


# speedup vs baseline: 1.0590x; 1.0590x over previous
import dataclasses
import functools

import jax
import jax.numpy as jnp
from jax import lax
from jax.experimental import pallas as pl
from jax.experimental.pallas import tpu as pltpu
from jax.experimental.pallas import tpu_sc as plsc

N = 10000
E = 320000
D = 128
H = 128
GATE_IN = D + 2 * H

NC = 2
NS = 16
NW = NC * NS
L = 16

B = 128
NCH = 80
NCH2 = 40
EPT = NCH * B
EPT2 = NCH2 * B
E_PAD = NW * EPT
TOTCH = E_PAD // B
NCHA = 120
PASSES_A = NCHA // NCH2
NCHB = 40
CPR = 624
TAIL = N - NS * CPR

@functools.cache
def _sc_kernels():
    mesh = plsc.VectorSubcoreMesh(core_axis_name="c", subcore_axis_name="s",
                                  num_cores=NC, num_subcores=NS)
    cp = pltpu.CompilerParams()
    if "needs_layout_passes" in pltpu.CompilerParams.__dataclass_fields__:
        cp = dataclasses.replace(cp, needs_layout_passes=False)
    deg = functools.partial(
        pl.kernel,
        compiler_params=cp,
        out_type=jax.ShapeDtypeStruct((NW, N), jnp.float32),
        mesh=mesh,
        scratch_types=[
            pltpu.VMEM((EPT,), jnp.int32),
            pltpu.VMEM((EPT,), jnp.float32),
            pltpu.VMEM((N,), jnp.float32),
        ],
    )(_deg_body)
    msg = functools.partial(
        pl.kernel,
        compiler_params=cp,
        out_type=jax.ShapeDtypeStruct((NC, N, H), jnp.float32),
        mesh=mesh,
        scratch_types=[
            pltpu.VMEM((NCH2, B), jnp.int32),
            pltpu.VMEM((NCH2, B), jnp.int32),
            pltpu.VMEM((EPT2,), jnp.float32),
            pltpu.VMEM((B, H), jnp.float32),
            pltpu.VMEM((B, H), jnp.float32),
            pltpu.VMEM_SHARED((N, H), jnp.float32),
            pltpu.SemaphoreType.DMA,
            pltpu.SemaphoreType.DMA,
            pltpu.SemaphoreType.DMA,
            pltpu.SemaphoreType.DMA,
        ],
    )(_msg_body)
    return deg, msg


def _deg_body(dst_hbm, ew_hbm, out_hbm, dst_v, ew_v, deg_v):
    cid = lax.axis_index("c")
    sid = lax.axis_index("s")
    wid = sid * NC + cid

    @pl.loop(0, N, step=L)
    def _(i):
        deg_v.at[pl.ds(i, L)][...] = jnp.zeros((L,), jnp.float32)

    pltpu.sync_copy(dst_hbm.at[wid], dst_v)
    pltpu.sync_copy(ew_hbm.at[wid], ew_v)

    @pl.loop(0, EPT, step=L)
    def _(i):
        idx = dst_v.at[pl.ds(i, L)][...]
        val = ew_v.at[pl.ds(i, L)][...]
        plsc.addupdate_scatter(deg_v, [idx], val)

    pltpu.sync_copy(deg_v, out_hbm.at[wid])


def _msg_body(xws_hbm, src_hbm, dst_hbm, ew_hbm, out_hbm,
              srcv, dstv, ewv, gbuf0, gbuf1, acc, gs0, gs1, ss0, ss1):
    cid = lax.axis_index("c")
    sid = lax.axis_index("s")
    wid = sid * NC + cid

    @pl.loop(0, B)
    def _(r):
        for j in range(H // L):
            gbuf0.at[r, pl.ds(j * L, L)][...] = jnp.zeros((L,), jnp.float32)

    r0 = sid * CPR
    for k in range(CPR // B):
        pltpu.sync_copy(gbuf0, acc.at[pl.ds(r0 + k * B, B)])
    rem = CPR - (CPR // B) * B
    pltpu.sync_copy(gbuf0.at[pl.ds(0, rem)],
                    acc.at[pl.ds(r0 + (CPR // B) * B, rem)])

    @pl.when(sid == 0)
    def _():
        pltpu.sync_copy(gbuf0.at[pl.ds(0, TAIL)],
                        acc.at[pl.ds(NS * CPR, TAIL)])

    def scale(buf, ch):
        @pl.loop(0, B, step=L)
        def _(rb):
            ews = ewv.at[pl.ds(ch * B + rb, L)][...]
            for k in range(L):
                s = ews[k]
                for j in range(H // L):
                    r = rb + k
                    buf.at[r, pl.ds(j * L, L)][...] = (
                        buf.at[r, pl.ds(j * L, L)][...] * s)

    def wait_gather(buf, sem):
        pltpu.make_async_copy(xws_hbm.at[srcv.at[0]], buf, sem).wait()

    def wait_scatter(sem):
        pltpu.make_async_copy(gbuf0, acc.at[dstv.at[0]], sem).wait()

    def do_pass(c0):
        pltpu.sync_copy(src_hbm.at[pl.ds(c0, NCH2)], srcv)
        pltpu.sync_copy(dst_hbm.at[pl.ds(c0, NCH2)], dstv)
        pltpu.sync_copy(ew_hbm.at[pl.ds(c0 * B, EPT2)], ewv)

        pltpu.async_copy(xws_hbm.at[srcv.at[0]], gbuf0, gs0)

        @pl.loop(0, NCH2, step=2)
        def _(i):
            @pl.when(i > 0)
            def _():
                wait_scatter(ss1)
            pltpu.async_copy(xws_hbm.at[srcv.at[i + 1]], gbuf1, gs1)
            wait_gather(gbuf0, gs0)
            scale(gbuf0, i)
            pltpu.async_copy(gbuf0, acc.at[dstv.at[i]], ss0, add=True)
            wait_gather(gbuf1, gs1)
            scale(gbuf1, i + 1)
            pltpu.async_copy(gbuf1, acc.at[dstv.at[i + 1]], ss1, add=True)

            @pl.when(i + 2 < NCH2)
            def _():
                wait_scatter(ss0)
                pltpu.async_copy(xws_hbm.at[srcv.at[i + 2]], gbuf0, gs0)

        wait_scatter(ss0)
        wait_scatter(ss1)

    plsc.subcore_barrier()

    @pl.when(cid == 0)
    def _():
        @pl.loop(0, PASSES_A)
        def _(p):
            do_pass(sid * NCHA + p * NCH2)

    @pl.when(cid == 1)
    def _():
        do_pass(NS * NCHA + sid * NCHB)

    plsc.subcore_barrier()
    pltpu.sync_copy(acc.at[pl.ds(r0, CPR)], out_hbm.at[cid, pl.ds(r0, CPR)])

    @pl.when(sid == 0)
    def _():
        pltpu.sync_copy(acc.at[pl.ds(NS * CPR, TAIL)],
                        out_hbm.at[cid, pl.ds(NS * CPR, TAIL)])


_BN = 1000


def _pre_body(x_ref, wt_ref, degp_ref, xws_ref, st_ref):
    xw = lax.dot_general(x_ref[...], wt_ref[...], (((1,), (0,)), ((), ())),
                         preferred_element_type=jnp.float32)
    deg = jnp.sum(degp_ref[...], axis=1) + 1.0
    dinv = lax.rsqrt(deg)
    xws_ref[...] = dinv[:, None] * xw
    st_ref[...] = (dinv * dinv)[:, None] * xw


def _pre_tc(x, w_t, degp):
    return pl.pallas_call(
        _pre_body,
        grid=(N // _BN,),
        in_specs=[
            pl.BlockSpec((_BN, D), lambda i: (i, 0)),
            pl.BlockSpec((D, H), lambda i: (0, 0)),
            pl.BlockSpec((_BN, NW), lambda i: (i, 0)),
        ],
        out_specs=[
            pl.BlockSpec((_BN, H), lambda i: (i, 0)),
            pl.BlockSpec((_BN, H), lambda i: (i, 0)),
        ],
        out_shape=[
            jax.ShapeDtypeStruct((N, H), jnp.float32),
            jax.ShapeDtypeStruct((N, H), jnp.float32),
        ],
    )(x, w_t, degp)


def _fin_body(x_ref, h_ref, c_ref, acc_ref, degp_ref, st_ref, bg_ref,
              wall_ref, ball_ref, hn_ref, cn_ref):
    accsum = acc_ref[0] + acc_ref[1]
    deg = jnp.sum(degp_ref[...], axis=1) + 1.0
    dinv = lax.rsqrt(deg)
    g = jax.nn.sigmoid(dinv[:, None] * accsum + st_ref[...] + bg_ref[...])
    zu = jnp.concatenate([x_ref[...], g, h_ref[...]], axis=1)
    pre = lax.dot_general(zu, wall_ref[...], (((1,), (0,)), ((), ())),
                          preferred_element_type=jnp.float32)
    pre = pre + ball_ref[...]
    f_t = jax.nn.sigmoid(pre[:, 0:H])
    i_t = jax.nn.sigmoid(pre[:, H:2 * H])
    o_t = jax.nn.sigmoid(pre[:, 2 * H:3 * H])
    c_t = jnp.tanh(pre[:, 3 * H:4 * H])
    c_next = f_t * c_ref[...] + i_t * c_t
    hn_ref[...] = o_t * jnp.tanh(c_next)
    cn_ref[...] = c_next


def _fin_tc(x, h, c, acc2, degp, st, bg, wall, ball):
    return pl.pallas_call(
        _fin_body,
        grid=(N // _BN,),
        in_specs=[
            pl.BlockSpec((_BN, D), lambda i: (i, 0)),
            pl.BlockSpec((_BN, H), lambda i: (i, 0)),
            pl.BlockSpec((_BN, H), lambda i: (i, 0)),
            pl.BlockSpec((NC, _BN, H), lambda i: (0, i, 0)),
            pl.BlockSpec((_BN, NW), lambda i: (i, 0)),
            pl.BlockSpec((_BN, H), lambda i: (i, 0)),
            pl.BlockSpec((1, H), lambda i: (0, 0)),
            pl.BlockSpec((GATE_IN, 4 * H), lambda i: (0, 0)),
            pl.BlockSpec((1, 4 * H), lambda i: (0, 0)),
        ],
        out_specs=[
            pl.BlockSpec((_BN, H), lambda i: (i, 0)),
            pl.BlockSpec((_BN, H), lambda i: (i, 0)),
        ],
        out_shape=[
            jax.ShapeDtypeStruct((N, H), jnp.float32),
            jax.ShapeDtypeStruct((N, H), jnp.float32),
        ],
    )(x, h, c, acc2, degp, st, bg, wall, ball)


def kernel(x, edge_index, edge_weight, h, c,
           W_gcn, b_gcn, W_f, b_f, W_i, b_i, W_o, b_o, W_c, b_c):
    src = edge_index[0]
    dst = edge_index[1]
    pad = E_PAD - E
    zpad_i = jnp.zeros((pad,), src.dtype)
    zpad_f = jnp.zeros((pad,), edge_weight.dtype)
    src_p = jnp.concatenate([src, zpad_i])
    dst_p = jnp.concatenate([dst, zpad_i])
    ew_p = jnp.concatenate([edge_weight, zpad_f])

    deg_sc, msg_sc = _sc_kernels()
    degp = deg_sc(dst_p.reshape(NW, EPT), ew_p.reshape(NW, EPT))
    degp = jnp.swapaxes(degp, 0, 1)
    xws, st = _pre_tc(x, W_gcn.T, degp)
    acc2 = msg_sc(xws, src_p.reshape(TOTCH, B), dst_p.reshape(TOTCH, B), ew_p)

    wall = jnp.concatenate([W_f.T, W_i.T, W_o.T, W_c.T], axis=1)
    ball = jnp.concatenate([b_f, b_i, b_o, b_c]).reshape(1, 4 * H)
    return _fin_tc(x, h, c, acc2, degp, st, b_gcn.reshape(1, H), wall, ball)


# speedup vs baseline: 1.0738x; 1.0140x over previous
import dataclasses
import functools

import jax
import jax.numpy as jnp
from jax import lax
from jax.experimental import pallas as pl
from jax.experimental.pallas import tpu as pltpu
from jax.experimental.pallas import tpu_sc as plsc

N = 10000
E = 320000
D = 128
H = 128
GATE_IN = D + 2 * H

NC = 2
NS = 16
NW = NC * NS
L = 16

B = 128
NCH = 80
NCH2 = 40
EPT = NCH * B
EPT2 = NCH2 * B
E_PAD = NW * EPT
TOTCH = E_PAD // B
NCHA = 120
PASSES_A = NCHA // NCH2
NCHB = 40
CPR = 624
TAIL = N - NS * CPR

@functools.cache
def _sc_kernels():
    mesh = plsc.VectorSubcoreMesh(core_axis_name="c", subcore_axis_name="s",
                                  num_cores=NC, num_subcores=NS)
    cp = pltpu.CompilerParams()
    if "needs_layout_passes" in pltpu.CompilerParams.__dataclass_fields__:
        cp = dataclasses.replace(cp, needs_layout_passes=False)
    deg = functools.partial(
        pl.kernel,
        compiler_params=cp,
        out_type=jax.ShapeDtypeStruct((NW, N), jnp.float32),
        mesh=mesh,
        scratch_types=[
            pltpu.VMEM((EPT,), jnp.int32),
            pltpu.VMEM((EPT,), jnp.float32),
            pltpu.VMEM((N,), jnp.float32),
        ],
    )(_deg_body)
    msg = functools.partial(
        pl.kernel,
        compiler_params=cp,
        out_type=jax.ShapeDtypeStruct((NC, N, H), jnp.float32),
        mesh=mesh,
        scratch_types=[
            pltpu.VMEM((NCH2, B), jnp.int32),
            pltpu.VMEM((NCH2, B), jnp.int32),
            pltpu.VMEM((EPT2,), jnp.float32),
            pltpu.VMEM((B, H), jnp.float32),
            pltpu.VMEM((B, H), jnp.float32),
            pltpu.VMEM_SHARED((N, H), jnp.float32),
            pltpu.SemaphoreType.DMA,
            pltpu.SemaphoreType.DMA,
            pltpu.SemaphoreType.DMA,
            pltpu.SemaphoreType.DMA,
        ],
    )(_msg_body)
    return deg, msg


def _deg_body(dst_hbm, ew_hbm, out_hbm, dst_v, ew_v, deg_v):
    cid = lax.axis_index("c")
    sid = lax.axis_index("s")
    wid = sid * NC + cid

    @pl.loop(0, N, step=L)
    def _(i):
        deg_v.at[pl.ds(i, L)][...] = jnp.zeros((L,), jnp.float32)

    pltpu.sync_copy(dst_hbm.at[wid], dst_v)
    pltpu.sync_copy(ew_hbm.at[wid], ew_v)

    @pl.loop(0, EPT, step=L)
    def _(i):
        idx = dst_v.at[pl.ds(i, L)][...]
        val = ew_v.at[pl.ds(i, L)][...]
        plsc.addupdate_scatter(deg_v, [idx], val)

    pltpu.sync_copy(deg_v, out_hbm.at[wid])


def _msg_body(xws_hbm, src_hbm, dst_hbm, ew_hbm, out_hbm,
              srcv, dstv, ewv, gbuf0, gbuf1, acc, gs0, gs1, ss0, ss1):
    cid = lax.axis_index("c")
    sid = lax.axis_index("s")
    wid = sid * NC + cid

    @pl.loop(0, B)
    def _(r):
        for j in range(H // L):
            gbuf0.at[r, pl.ds(j * L, L)][...] = jnp.zeros((L,), jnp.float32)

    r0 = sid * CPR
    for k in range(CPR // B):
        pltpu.sync_copy(gbuf0, acc.at[pl.ds(r0 + k * B, B)])
    rem = CPR - (CPR // B) * B
    pltpu.sync_copy(gbuf0.at[pl.ds(0, rem)],
                    acc.at[pl.ds(r0 + (CPR // B) * B, rem)])

    @pl.when(sid == 0)
    def _():
        pltpu.sync_copy(gbuf0.at[pl.ds(0, TAIL)],
                        acc.at[pl.ds(NS * CPR, TAIL)])

    def scale(buf, ch):
        @pl.loop(0, B, step=L)
        def _(rb):
            ews = ewv.at[pl.ds(ch * B + rb, L)][...]
            for k in range(L):
                s = ews[k]
                for j in range(H // L):
                    r = rb + k
                    buf.at[r, pl.ds(j * L, L)][...] = (
                        buf.at[r, pl.ds(j * L, L)][...] * s)

    def wait_gather(buf, sem):
        pltpu.make_async_copy(xws_hbm.at[srcv.at[0]], buf, sem).wait()

    def wait_scatter(sem):
        pltpu.make_async_copy(gbuf0, acc.at[dstv.at[0]], sem).wait()

    def do_pass(c0):
        pltpu.sync_copy(src_hbm.at[pl.ds(c0, NCH2)], srcv)
        pltpu.sync_copy(dst_hbm.at[pl.ds(c0, NCH2)], dstv)
        pltpu.sync_copy(ew_hbm.at[pl.ds(c0 * B, EPT2)], ewv)

        pltpu.async_copy(xws_hbm.at[srcv.at[0]], gbuf0, gs0)

        @pl.loop(0, NCH2, step=2)
        def _(i):
            @pl.when(i > 0)
            def _():
                wait_scatter(ss1)
            pltpu.async_copy(xws_hbm.at[srcv.at[i + 1]], gbuf1, gs1)
            wait_gather(gbuf0, gs0)
            scale(gbuf0, i)
            pltpu.async_copy(gbuf0, acc.at[dstv.at[i]], ss0, add=True)
            wait_gather(gbuf1, gs1)
            scale(gbuf1, i + 1)
            pltpu.async_copy(gbuf1, acc.at[dstv.at[i + 1]], ss1, add=True)

            @pl.when(i + 2 < NCH2)
            def _():
                wait_scatter(ss0)
                pltpu.async_copy(xws_hbm.at[srcv.at[i + 2]], gbuf0, gs0)

        wait_scatter(ss0)
        wait_scatter(ss1)

    plsc.subcore_barrier()

    start = jnp.where(cid == 0, sid * NCHA, NS * NCHA + sid * NCHB)
    npass = jnp.where(cid == 0, NCHA // NCH2, NCHB // NCH2)

    @pl.loop(0, npass)
    def _(p):
        do_pass(start + p * NCH2)

    plsc.subcore_barrier()
    pltpu.sync_copy(acc.at[pl.ds(r0, CPR)], out_hbm.at[cid, pl.ds(r0, CPR)])

    @pl.when(sid == 0)
    def _():
        pltpu.sync_copy(acc.at[pl.ds(NS * CPR, TAIL)],
                        out_hbm.at[cid, pl.ds(NS * CPR, TAIL)])


_BN = 1000


def _pre_body(x_ref, wt_ref, degp_ref, xws_ref, st_ref):
    xw = lax.dot_general(x_ref[...], wt_ref[...], (((1,), (0,)), ((), ())),
                         preferred_element_type=jnp.float32)
    deg = jnp.sum(degp_ref[...], axis=1) + 1.0
    dinv = lax.rsqrt(deg)
    xws_ref[...] = dinv[:, None] * xw
    st_ref[...] = (dinv * dinv)[:, None] * xw


def _pre_tc(x, w_t, degp):
    return pl.pallas_call(
        _pre_body,
        grid=(N // _BN,),
        in_specs=[
            pl.BlockSpec((_BN, D), lambda i: (i, 0)),
            pl.BlockSpec((D, H), lambda i: (0, 0)),
            pl.BlockSpec((_BN, NW), lambda i: (i, 0)),
        ],
        out_specs=[
            pl.BlockSpec((_BN, H), lambda i: (i, 0)),
            pl.BlockSpec((_BN, H), lambda i: (i, 0)),
        ],
        out_shape=[
            jax.ShapeDtypeStruct((N, H), jnp.float32),
            jax.ShapeDtypeStruct((N, H), jnp.float32),
        ],
    )(x, w_t, degp)


def _fin_body(x_ref, h_ref, c_ref, acc_ref, degp_ref, st_ref, bg_ref,
              wall_ref, ball_ref, hn_ref, cn_ref):
    accsum = acc_ref[0] + acc_ref[1]
    deg = jnp.sum(degp_ref[...], axis=1) + 1.0
    dinv = lax.rsqrt(deg)
    g = jax.nn.sigmoid(dinv[:, None] * accsum + st_ref[...] + bg_ref[...])
    zu = jnp.concatenate([x_ref[...], g, h_ref[...]], axis=1)
    pre = lax.dot_general(zu, wall_ref[...], (((1,), (0,)), ((), ())),
                          preferred_element_type=jnp.float32)
    pre = pre + ball_ref[...]
    f_t = jax.nn.sigmoid(pre[:, 0:H])
    i_t = jax.nn.sigmoid(pre[:, H:2 * H])
    o_t = jax.nn.sigmoid(pre[:, 2 * H:3 * H])
    c_t = jnp.tanh(pre[:, 3 * H:4 * H])
    c_next = f_t * c_ref[...] + i_t * c_t
    hn_ref[...] = o_t * jnp.tanh(c_next)
    cn_ref[...] = c_next


def _fin_tc(x, h, c, acc2, degp, st, bg, wall, ball):
    return pl.pallas_call(
        _fin_body,
        grid=(N // _BN,),
        in_specs=[
            pl.BlockSpec((_BN, D), lambda i: (i, 0)),
            pl.BlockSpec((_BN, H), lambda i: (i, 0)),
            pl.BlockSpec((_BN, H), lambda i: (i, 0)),
            pl.BlockSpec((NC, _BN, H), lambda i: (0, i, 0)),
            pl.BlockSpec((_BN, NW), lambda i: (i, 0)),
            pl.BlockSpec((_BN, H), lambda i: (i, 0)),
            pl.BlockSpec((1, H), lambda i: (0, 0)),
            pl.BlockSpec((GATE_IN, 4 * H), lambda i: (0, 0)),
            pl.BlockSpec((1, 4 * H), lambda i: (0, 0)),
        ],
        out_specs=[
            pl.BlockSpec((_BN, H), lambda i: (i, 0)),
            pl.BlockSpec((_BN, H), lambda i: (i, 0)),
        ],
        out_shape=[
            jax.ShapeDtypeStruct((N, H), jnp.float32),
            jax.ShapeDtypeStruct((N, H), jnp.float32),
        ],
    )(x, h, c, acc2, degp, st, bg, wall, ball)


def kernel(x, edge_index, edge_weight, h, c,
           W_gcn, b_gcn, W_f, b_f, W_i, b_i, W_o, b_o, W_c, b_c):
    src = edge_index[0]
    dst = edge_index[1]
    pad = E_PAD - E
    zpad_i = jnp.zeros((pad,), src.dtype)
    zpad_f = jnp.zeros((pad,), edge_weight.dtype)
    src_p = jnp.concatenate([src, zpad_i])
    dst_p = jnp.concatenate([dst, zpad_i])
    ew_p = jnp.concatenate([edge_weight, zpad_f])

    deg_sc, msg_sc = _sc_kernels()
    degp = deg_sc(dst_p.reshape(NW, EPT), ew_p.reshape(NW, EPT))
    degp = jnp.swapaxes(degp, 0, 1)
    xws, st = _pre_tc(x, W_gcn.T, degp)
    acc2 = msg_sc(xws, src_p.reshape(TOTCH, B), dst_p.reshape(TOTCH, B), ew_p)

    wall = jnp.concatenate([W_f.T, W_i.T, W_o.T, W_c.T], axis=1)
    ball = jnp.concatenate([b_f, b_i, b_o, b_c]).reshape(1, 4 * H)
    return _fin_tc(x, h, c, acc2, degp, st, b_gcn.reshape(1, H), wall, ball)


# speedup vs baseline: 1.1200x; 1.0430x over previous
import dataclasses
import functools

import jax
import jax.numpy as jnp
from jax import lax
from jax.experimental import pallas as pl
from jax.experimental.pallas import tpu as pltpu
from jax.experimental.pallas import tpu_sc as plsc

N = 10000
E = 320000
D = 128
H = 128
GATE_IN = D + 2 * H

NC = 2
NS = 16
NW = NC * NS
L = 16

B = 64
HP = H // 2
EPT = 10240
E_PAD = NW * EPT
TOTCH = E_PAD // B
NCH2 = 64
EPT2 = NCH2 * B
NCHA = 256
NCHB = 64
CPR = 624
TAIL = N - NS * CPR

@functools.cache
def _sc_kernels():
    mesh = plsc.VectorSubcoreMesh(core_axis_name="c", subcore_axis_name="s",
                                  num_cores=NC, num_subcores=NS)
    cp = pltpu.CompilerParams()
    if "needs_layout_passes" in pltpu.CompilerParams.__dataclass_fields__:
        cp = dataclasses.replace(cp, needs_layout_passes=False)
    cp = dataclasses.replace(cp, use_tc_tiling_on_sc=False)
    deg = functools.partial(
        pl.kernel,
        compiler_params=cp,
        out_type=jax.ShapeDtypeStruct((NW, N), jnp.float32),
        mesh=mesh,
        scratch_types=[
            pltpu.VMEM((EPT,), jnp.int32),
            pltpu.VMEM((EPT,), jnp.float32),
            pltpu.VMEM((N,), jnp.float32),
        ],
    )(_deg_body)
    msg = functools.partial(
        pl.kernel,
        compiler_params=cp,
        out_type=jax.ShapeDtypeStruct((NC, N, H), jnp.float32),
        mesh=mesh,
        scratch_types=[
            pltpu.VMEM((NCH2, B), jnp.int32),
            pltpu.VMEM((NCH2, B), jnp.int32),
            pltpu.VMEM((EPT2,), jnp.float32),
            pltpu.VMEM((B, HP), jnp.int32),
            pltpu.VMEM((B, HP), jnp.int32),
            pltpu.VMEM((B, H), jnp.float32),
            pltpu.VMEM((B, H), jnp.float32),
            pltpu.VMEM_SHARED((N, H), jnp.float32),
            pltpu.SemaphoreType.DMA,
            pltpu.SemaphoreType.DMA,
            pltpu.SemaphoreType.DMA,
            pltpu.SemaphoreType.DMA,
        ],
    )(_msg_body)
    return deg, msg


def _deg_body(dst_hbm, ew_hbm, out_hbm, dst_v, ew_v, deg_v):
    cid = lax.axis_index("c")
    sid = lax.axis_index("s")
    wid = sid * NC + cid

    @pl.loop(0, N, step=L)
    def _(i):
        deg_v.at[pl.ds(i, L)][...] = jnp.zeros((L,), jnp.float32)

    pltpu.sync_copy(dst_hbm.at[wid], dst_v)
    pltpu.sync_copy(ew_hbm.at[wid], ew_v)

    @pl.loop(0, EPT, step=L)
    def _(i):
        idx = dst_v.at[pl.ds(i, L)][...]
        val = ew_v.at[pl.ds(i, L)][...]
        plsc.addupdate_scatter(deg_v, [idx], val)

    pltpu.sync_copy(deg_v, out_hbm.at[wid])


def _msg_body(xwp_hbm, src_hbm, dst_hbm, ew_hbm, out_hbm,
              srcv, dstv, ewv, pbuf0, pbuf1, sbuf0, sbuf1, acc,
              gs0, gs1, ss0, ss1):
    cid = lax.axis_index("c")
    sid = lax.axis_index("s")

    @pl.loop(0, B)
    def _(r):
        for j in range(H // L):
            sbuf0.at[r, pl.ds(j * L, L)][...] = jnp.zeros((L,), jnp.float32)

    r0 = sid * CPR
    for k in range(CPR // B):
        pltpu.sync_copy(sbuf0, acc.at[pl.ds(r0 + k * B, B)])
    rem = CPR - (CPR // B) * B
    pltpu.sync_copy(sbuf0.at[pl.ds(0, rem)],
                    acc.at[pl.ds(r0 + (CPR // B) * B, rem)])

    @pl.when(sid == 0)
    def _():
        pltpu.sync_copy(sbuf0.at[pl.ds(0, TAIL)],
                        acc.at[pl.ds(NS * CPR, TAIL)])

    msk = jnp.full((L,), -65536, jnp.int32)

    def unpack_scale(pb, sb, ch):
        @pl.loop(0, B, step=L)
        def _(rb):
            ews = ewv.at[pl.ds(ch * B + rb, L)][...]
            for k in range(L):
                s = ews[k]
                r = rb + k
                for j in range(HP // L):
                    v = pb.at[r, pl.ds(j * L, L)][...]
                    flo = plsc.bitcast(lax.shift_left(v, 16), jnp.float32)
                    fhi = plsc.bitcast(lax.bitwise_and(v, msk), jnp.float32)
                    sb.at[r, pl.ds(j * L, L)][...] = flo * s
                    sb.at[r, pl.ds(HP + j * L, L)][...] = fhi * s

    def gath(i, pb, sem):
        pltpu.async_copy(xwp_hbm.at[srcv.at[i]], pb, sem)

    def scat(i, sb, sem):
        pltpu.async_copy(sb, acc.at[dstv.at[i]], sem, add=True)

    def wait_gather(pb, sem):
        pltpu.make_async_copy(xwp_hbm.at[srcv.at[0]], pb, sem).wait()

    def wait_scatter(sb, sem):
        pltpu.make_async_copy(sb, acc.at[dstv.at[0]], sem).wait()

    def do_pass(c0):
        pltpu.sync_copy(src_hbm.at[pl.ds(c0, NCH2)], srcv)
        pltpu.sync_copy(dst_hbm.at[pl.ds(c0, NCH2)], dstv)
        pltpu.sync_copy(ew_hbm.at[pl.ds(c0 * B, EPT2)], ewv)

        gath(0, pbuf0, gs0)
        gath(1, pbuf1, gs1)

        @pl.loop(0, NCH2, step=2)
        def _(i):
            wait_gather(pbuf0, gs0)

            @pl.when(i > 0)
            def _():
                wait_scatter(sbuf0, ss0)
            unpack_scale(pbuf0, sbuf0, i)

            @pl.when(i + 2 < NCH2)
            def _():
                gath(i + 2, pbuf0, gs0)
            scat(i, sbuf0, ss0)
            wait_gather(pbuf1, gs1)

            @pl.when(i > 0)
            def _():
                wait_scatter(sbuf1, ss1)
            unpack_scale(pbuf1, sbuf1, i + 1)

            @pl.when(i + 3 < NCH2)
            def _():
                gath(i + 3, pbuf1, gs1)
            scat(i + 1, sbuf1, ss1)

        wait_scatter(sbuf0, ss0)
        wait_scatter(sbuf1, ss1)

    plsc.subcore_barrier()

    start = jnp.where(cid == 0, sid * NCHA, NS * NCHA + sid * NCHB)
    npass = jnp.where(cid == 0, NCHA // NCH2, NCHB // NCH2)

    @pl.loop(0, npass)
    def _(p):
        do_pass(start + p * NCH2)

    plsc.subcore_barrier()
    pltpu.sync_copy(acc.at[pl.ds(r0, CPR)], out_hbm.at[cid, pl.ds(r0, CPR)])

    @pl.when(sid == 0)
    def _():
        pltpu.sync_copy(acc.at[pl.ds(NS * CPR, TAIL)],
                        out_hbm.at[cid, pl.ds(NS * CPR, TAIL)])


_BN = 1000


def _pre_body(x_ref, wt_ref, degp_ref, xwp_ref, st_ref):
    xw = lax.dot_general(x_ref[...], wt_ref[...], (((1,), (0,)), ((), ())),
                         preferred_element_type=jnp.float32)
    deg = jnp.sum(degp_ref[...], axis=1) + 1.0
    dinv = lax.rsqrt(deg)
    xws = dinv[:, None] * xw
    xb = xws.astype(jnp.bfloat16)
    lo = lax.bitcast_convert_type(xb[:, :HP], jnp.uint16).astype(jnp.uint32)
    hi = lax.bitcast_convert_type(xb[:, HP:], jnp.uint16).astype(jnp.uint32)
    xwp_ref[...] = lax.bitcast_convert_type(
        lax.shift_left(hi, jnp.uint32(16)) | lo, jnp.int32)
    st_ref[...] = (dinv * dinv)[:, None] * xw


def _pre_tc(x, w_t, degp):
    return pl.pallas_call(
        _pre_body,
        grid=(N // _BN,),
        in_specs=[
            pl.BlockSpec((_BN, D), lambda i: (i, 0)),
            pl.BlockSpec((D, H), lambda i: (0, 0)),
            pl.BlockSpec((_BN, NW), lambda i: (i, 0)),
        ],
        out_specs=[
            pl.BlockSpec((_BN, HP), lambda i: (i, 0)),
            pl.BlockSpec((_BN, H), lambda i: (i, 0)),
        ],
        out_shape=[
            jax.ShapeDtypeStruct((N, HP), jnp.int32),
            jax.ShapeDtypeStruct((N, H), jnp.float32),
        ],
    )(x, w_t, degp)


def _fin_body(x_ref, h_ref, c_ref, acc_ref, degp_ref, st_ref, bg_ref,
              wall_ref, ball_ref, hn_ref, cn_ref):
    accsum = acc_ref[0] + acc_ref[1]
    deg = jnp.sum(degp_ref[...], axis=1) + 1.0
    dinv = lax.rsqrt(deg)
    g = jax.nn.sigmoid(dinv[:, None] * accsum + st_ref[...] + bg_ref[...])
    zu = jnp.concatenate([x_ref[...], g, h_ref[...]], axis=1)
    pre = lax.dot_general(zu, wall_ref[...], (((1,), (0,)), ((), ())),
                          preferred_element_type=jnp.float32)
    pre = pre + ball_ref[...]
    f_t = jax.nn.sigmoid(pre[:, 0:H])
    i_t = jax.nn.sigmoid(pre[:, H:2 * H])
    o_t = jax.nn.sigmoid(pre[:, 2 * H:3 * H])
    c_t = jnp.tanh(pre[:, 3 * H:4 * H])
    c_next = f_t * c_ref[...] + i_t * c_t
    hn_ref[...] = o_t * jnp.tanh(c_next)
    cn_ref[...] = c_next


def _fin_tc(x, h, c, acc2, degp, st, bg, wall, ball):
    return pl.pallas_call(
        _fin_body,
        grid=(N // _BN,),
        in_specs=[
            pl.BlockSpec((_BN, D), lambda i: (i, 0)),
            pl.BlockSpec((_BN, H), lambda i: (i, 0)),
            pl.BlockSpec((_BN, H), lambda i: (i, 0)),
            pl.BlockSpec((NC, _BN, H), lambda i: (0, i, 0)),
            pl.BlockSpec((_BN, NW), lambda i: (i, 0)),
            pl.BlockSpec((_BN, H), lambda i: (i, 0)),
            pl.BlockSpec((1, H), lambda i: (0, 0)),
            pl.BlockSpec((GATE_IN, 4 * H), lambda i: (0, 0)),
            pl.BlockSpec((1, 4 * H), lambda i: (0, 0)),
        ],
        out_specs=[
            pl.BlockSpec((_BN, H), lambda i: (i, 0)),
            pl.BlockSpec((_BN, H), lambda i: (i, 0)),
        ],
        out_shape=[
            jax.ShapeDtypeStruct((N, H), jnp.float32),
            jax.ShapeDtypeStruct((N, H), jnp.float32),
        ],
    )(x, h, c, acc2, degp, st, bg, wall, ball)


def kernel(x, edge_index, edge_weight, h, c,
           W_gcn, b_gcn, W_f, b_f, W_i, b_i, W_o, b_o, W_c, b_c):
    src = edge_index[0]
    dst = edge_index[1]
    pad = E_PAD - E
    zpad_i = jnp.zeros((pad,), src.dtype)
    zpad_f = jnp.zeros((pad,), edge_weight.dtype)
    src_p = jnp.concatenate([src, zpad_i])
    dst_p = jnp.concatenate([dst, zpad_i])
    ew_p = jnp.concatenate([edge_weight, zpad_f])

    deg_sc, msg_sc = _sc_kernels()
    degp = deg_sc(dst_p.reshape(NW, EPT), ew_p.reshape(NW, EPT))
    degp = jnp.swapaxes(degp, 0, 1)
    xwp, st = _pre_tc(x, W_gcn.T, degp)
    acc2 = msg_sc(xwp, src_p.reshape(TOTCH, B), dst_p.reshape(TOTCH, B), ew_p)

    wall = jnp.concatenate([W_f.T, W_i.T, W_o.T, W_c.T], axis=1)
    ball = jnp.concatenate([b_f, b_i, b_o, b_c]).reshape(1, 4 * H)
    return _fin_tc(x, h, c, acc2, degp, st, b_gcn.reshape(1, H), wall, ball)
